# baseline (speedup 1.0000x reference)
import functools

import jax
import jax.numpy as jnp
from jax import lax
from jax.experimental import pallas as pl
from jax.experimental.pallas import tpu as pltpu
from jax.experimental.pallas import tpu_sc as plsc

NC = 2
NS = 16
NW = NC * NS
L = 16

B = 16384
V = 1000000
R = 8
D = 64
MARGIN = 1.0

S = B // NW
CHUNK = 128
NCHUNK = S // CHUNK
GROUPS = CHUNK // L


def _body(ph_i, pt_i, nh_i, nt_i, pr_i, nr_i, pw_i, nw_i, ent, rel,
          out_hbm,
          ih_v, it_v, jh_v, jt_v, pr_v, nr_v, pw_v, nw_v, rel_v,
          ph0, pt0, nh0, nt0, ph1, pt1, nh1, nt1,
          out_v, sem0, sem1):
    cid = lax.axis_index("c")
    sid = lax.axis_index("s")
    wid = sid * NC + cid

    pltpu.sync_copy(ph_i.at[wid], ih_v)
    pltpu.sync_copy(pt_i.at[wid], it_v)
    pltpu.sync_copy(nh_i.at[wid], jh_v)
    pltpu.sync_copy(nt_i.at[wid], jt_v)
    pltpu.sync_copy(pr_i.at[wid], pr_v)
    pltpu.sync_copy(nr_i.at[wid], nr_v)
    pltpu.sync_copy(pw_i.at[wid], pw_v)
    pltpu.sync_copy(nw_i.at[wid], nw_v)
    pltpu.sync_copy(rel, rel_v)

    bufs = ((ph0, pt0, nh0, nt0), (ph1, pt1, nh1, nt1))
    sems = (sem0, sem1)

    def fire(c):
        p = c % 2
        return [
            pltpu.async_copy(ent.at[ih_v.at[c]], bufs[p][0], sems[p]),
            pltpu.async_copy(ent.at[it_v.at[c]], bufs[p][1], sems[p]),
            pltpu.async_copy(ent.at[jh_v.at[c]], bufs[p][2], sems[p]),
            pltpu.async_copy(ent.at[jt_v.at[c]], bufs[p][3], sems[p]),
        ]

    pending = fire(0)
    total = jnp.zeros((L,), jnp.float32)
    for c in range(NCHUNK):
        nxt = fire(c + 1) if c + 1 < NCHUNK else []
        for dsc in pending:
            dsc.wait()
        phb, ptb, nhb, ntb = bufs[c % 2]
        for g in range(GROUPS):
            off = c * CHUNK + g * L
            rows = lax.iota(jnp.int32, L) + (g * L)
            prv = pr_v[pl.ds(off, L)]
            nrv = nr_v[pl.ds(off, L)]
            pw = pw_v[pl.ds(off, L)]
            nw = nw_v[pl.ds(off, L)]

            def dbody(_, carry, rows=rows, prv=prv, nrv=nrv,
                      phb=phb, ptb=ptb, nhb=nhb, ntb=ntb):
                dvec, ap, an = carry
                hp = plsc.load_gather(phb, [rows, dvec])
                rp = plsc.load_gather(rel_v, [prv, dvec])
                tp = plsc.load_gather(ptb, [rows, dvec])
                hn = plsc.load_gather(nhb, [rows, dvec])
                rn = plsc.load_gather(rel_v, [nrv, dvec])
                tn = plsc.load_gather(ntb, [rows, dvec])
                ap = ap + jnp.abs(hp + rp - tp)
                an = an + jnp.abs(hn + rn - tn)
                return (dvec + 1, ap, an)

            zf = jnp.zeros((L,), jnp.float32)
            zi = jnp.zeros((L,), jnp.int32)
            _, ps, ns = lax.fori_loop(0, D, dbody, (zi, zf, zf))
            total = total + jnp.maximum(pw * ps - nw * ns + MARGIN, 0.0)
        pending = nxt

    out_v[...] = total
    pltpu.sync_copy(out_v, out_hbm.at[wid])


_rhine_sc = functools.partial(
    pl.kernel,
    out_type=jax.ShapeDtypeStruct((NW, L), jnp.float32),
    mesh=plsc.VectorSubcoreMesh(core_axis_name="c", subcore_axis_name="s"),
    compiler_params=pltpu.CompilerParams(
        needs_layout_passes=False, use_tc_tiling_on_sc=False),
    scratch_types=[
        pltpu.VMEM((NCHUNK, CHUNK), jnp.int32),
        pltpu.VMEM((NCHUNK, CHUNK), jnp.int32),
        pltpu.VMEM((NCHUNK, CHUNK), jnp.int32),
        pltpu.VMEM((NCHUNK, CHUNK), jnp.int32),
        pltpu.VMEM((S,), jnp.int32),
        pltpu.VMEM((S,), jnp.int32),
        pltpu.VMEM((S,), jnp.float32),
        pltpu.VMEM((S,), jnp.float32),
        pltpu.VMEM((R, D), jnp.float32),
        pltpu.VMEM((CHUNK, D), jnp.float32),
        pltpu.VMEM((CHUNK, D), jnp.float32),
        pltpu.VMEM((CHUNK, D), jnp.float32),
        pltpu.VMEM((CHUNK, D), jnp.float32),
        pltpu.VMEM((CHUNK, D), jnp.float32),
        pltpu.VMEM((CHUNK, D), jnp.float32),
        pltpu.VMEM((CHUNK, D), jnp.float32),
        pltpu.VMEM((CHUNK, D), jnp.float32),
        pltpu.VMEM((L,), jnp.float32),
        pltpu.SemaphoreType.DMA,
        pltpu.SemaphoreType.DMA,
    ],
)(_body)


@jax.jit
def _run(pos_h, pos_t, pos_r, pos_w, neg_h, neg_t, neg_r, neg_w,
         ent_emb, rel_emb):
    i32 = lambda x: x.astype(jnp.int32)
    ph = i32(pos_h).reshape(NW, NCHUNK, CHUNK)
    pt = i32(pos_t).reshape(NW, NCHUNK, CHUNK)
    nh = i32(neg_h).reshape(NW, NCHUNK, CHUNK)
    nt = i32(neg_t).reshape(NW, NCHUNK, CHUNK)
    pr = i32(pos_r).reshape(NW, S)
    nr = i32(neg_r).reshape(NW, S)
    pw = pos_w.astype(jnp.float32).reshape(NW, S)
    nw = neg_w.astype(jnp.float32).reshape(NW, S)
    out = _rhine_sc(ph, pt, nh, nt, pr, nr, pw, nw,
                    ent_emb.astype(jnp.float32), rel_emb.astype(jnp.float32))
    return jnp.sum(out)


def kernel(pos_h, pos_t, pos_r, pos_w, neg_h, neg_t, neg_r, neg_w,
           ent_emb, rel_emb):
    return _run(pos_h, pos_t, pos_r, pos_w, neg_h, neg_t, neg_r, neg_w,
                ent_emb, rel_emb)


# speedup vs baseline: 1.0209x; 1.0209x over previous
import functools

import jax
import jax.numpy as jnp
from jax import lax
from jax.experimental import pallas as pl
from jax.experimental.pallas import tpu as pltpu
from jax.experimental.pallas import tpu_sc as plsc

NC = 2
NS = 16
NW = NC * NS
L = 16

B = 16384
V = 1000000
R = 8
D = 64
W = 2 * D
MARGIN = 1.0

S = B // NW
CHUNK = 64
NCHUNK = S // CHUNK
GROUPS = CHUNK // L
UNROLL = 4


def _body(hidx, tidx, jhidx, jtidx, bh, bt, bjh, bjt, prb, nrb, pw, nw,
          ent2, relf,
          out_hbm,
          hidx_v, tidx_v, jhidx_v, jtidx_v,
          bh_v, bt_v, bjh_v, bjt_v,
          prb_v, nrb_v, pw_v, nw_v, relf_v,
          ph0, pt0, nh0, nt0, ph1, pt1, nh1, nt1,
          out_v, sem0, sem1):
    cid = lax.axis_index("c")
    sid = lax.axis_index("s")
    wid = sid * NC + cid
    base = wid * S

    pltpu.sync_copy(hidx.at[pl.ds(base, S)], hidx_v)
    pltpu.sync_copy(tidx.at[pl.ds(base, S)], tidx_v)
    pltpu.sync_copy(jhidx.at[pl.ds(base, S)], jhidx_v)
    pltpu.sync_copy(jtidx.at[pl.ds(base, S)], jtidx_v)
    pltpu.sync_copy(bh.at[pl.ds(base, S)], bh_v)
    pltpu.sync_copy(bt.at[pl.ds(base, S)], bt_v)
    pltpu.sync_copy(bjh.at[pl.ds(base, S)], bjh_v)
    pltpu.sync_copy(bjt.at[pl.ds(base, S)], bjt_v)
    pltpu.sync_copy(prb.at[pl.ds(base, S)], prb_v)
    pltpu.sync_copy(nrb.at[pl.ds(base, S)], nrb_v)
    pltpu.sync_copy(pw.at[pl.ds(base, S)], pw_v)
    pltpu.sync_copy(nw.at[pl.ds(base, S)], nw_v)
    pltpu.sync_copy(relf, relf_v)

    bufs = ((ph0, pt0, nh0, nt0), (ph1, pt1, nh1, nt1))
    sems = (sem0, sem1)
    idxs = (hidx_v, tidx_v, jhidx_v, jtidx_v)

    def fire(c):
        p = c % 2
        return [
            pltpu.async_copy(ent2.at[idxs[k].at[pl.ds(c * CHUNK, CHUNK)]],
                             bufs[p][k], sems[p])
            for k in range(4)
        ]

    pending = fire(0)
    total = jnp.zeros((L,), jnp.float32)
    for c in range(NCHUNK):
        nxt = fire(c + 1) if c + 1 < NCHUNK else []
        for dsc in pending:
            dsc.wait()
        phb, ptb, nhb, ntb = bufs[c % 2]
        for g in range(GROUPS):
            off = c * CHUNK + g * L
            rows = lax.iota(jnp.int32, L) + (g * L)
            eh = bh_v[pl.ds(off, L)]
            et = bt_v[pl.ds(off, L)]
            ejh = bjh_v[pl.ds(off, L)]
            ejt = bjt_v[pl.ds(off, L)]
            cp = prb_v[pl.ds(off, L)]
            cn = nrb_v[pl.ds(off, L)]
            pwv = pw_v[pl.ds(off, L)]
            nwv = nw_v[pl.ds(off, L)]

            def dbody(_, carry, phb=phb, ptb=ptb, nhb=nhb, ntb=ntb,
                      rows=rows):
                i0, i1, i2, i3, i4, i5, ap, an = carry
                for _u in range(UNROLL):
                    hp = plsc.load_gather(phb, [rows, i0])
                    tp = plsc.load_gather(ptb, [rows, i1])
                    hn = plsc.load_gather(nhb, [rows, i2])
                    tn = plsc.load_gather(ntb, [rows, i3])
                    rp = plsc.load_gather(relf_v, [i4])
                    rn = plsc.load_gather(relf_v, [i5])
                    ap = ap + jnp.abs(hp + rp - tp)
                    an = an + jnp.abs(hn + rn - tn)
                    i0 = i0 + 1
                    i1 = i1 + 1
                    i2 = i2 + 1
                    i3 = i3 + 1
                    i4 = i4 + 1
                    i5 = i5 + 1
                return (i0, i1, i2, i3, i4, i5, ap, an)

            zf = jnp.zeros((L,), jnp.float32)
            out = lax.fori_loop(0, D // UNROLL, dbody,
                                (eh, et, ejh, ejt, cp, cn, zf, zf))
            ps, ns = out[6], out[7]
            total = total + jnp.maximum(pwv * ps - nwv * ns + MARGIN, 0.0)
        pending = nxt

    out_v[...] = total
    pltpu.sync_copy(out_v, out_hbm.at[pl.ds(wid * L, L)])


_rhine_sc = functools.partial(
    pl.kernel,
    out_type=jax.ShapeDtypeStruct((NW * L,), jnp.float32),
    mesh=plsc.VectorSubcoreMesh(core_axis_name="c", subcore_axis_name="s"),
    compiler_params=pltpu.CompilerParams(
        needs_layout_passes=False, use_tc_tiling_on_sc=True),
    scratch_types=[
        pltpu.VMEM((S,), jnp.int32),
        pltpu.VMEM((S,), jnp.int32),
        pltpu.VMEM((S,), jnp.int32),
        pltpu.VMEM((S,), jnp.int32),
        pltpu.VMEM((S,), jnp.int32),
        pltpu.VMEM((S,), jnp.int32),
        pltpu.VMEM((S,), jnp.int32),
        pltpu.VMEM((S,), jnp.int32),
        pltpu.VMEM((S,), jnp.int32),
        pltpu.VMEM((S,), jnp.int32),
        pltpu.VMEM((S,), jnp.float32),
        pltpu.VMEM((S,), jnp.float32),
        pltpu.VMEM((R * D,), jnp.float32),
        pltpu.VMEM((CHUNK, W), jnp.float32),
        pltpu.VMEM((CHUNK, W), jnp.float32),
        pltpu.VMEM((CHUNK, W), jnp.float32),
        pltpu.VMEM((CHUNK, W), jnp.float32),
        pltpu.VMEM((CHUNK, W), jnp.float32),
        pltpu.VMEM((CHUNK, W), jnp.float32),
        pltpu.VMEM((CHUNK, W), jnp.float32),
        pltpu.VMEM((CHUNK, W), jnp.float32),
        pltpu.VMEM((L,), jnp.float32),
        pltpu.SemaphoreType.DMA,
        pltpu.SemaphoreType.DMA,
    ],
)(_body)


@jax.jit
def _run(pos_h, pos_t, pos_r, pos_w, neg_h, neg_t, neg_r, neg_w,
         ent_emb, rel_emb):
    i32 = lambda x: x.astype(jnp.int32)
    ph, pt, nh, nt = i32(pos_h), i32(pos_t), i32(neg_h), i32(neg_t)
    out = _rhine_sc(
        ph >> 1, pt >> 1, nh >> 1, nt >> 1,
        (ph & 1) * D, (pt & 1) * D,
        (nh & 1) * D, (nt & 1) * D,
        i32(pos_r) * D, i32(neg_r) * D,
        pos_w.astype(jnp.float32), neg_w.astype(jnp.float32),
        ent_emb.astype(jnp.float32).reshape(V // 2, W),
        rel_emb.astype(jnp.float32).reshape(R * D),
    )
    return jnp.sum(out)


def kernel(pos_h, pos_t, pos_r, pos_w, neg_h, neg_t, neg_r, neg_w,
           ent_emb, rel_emb):
    return _run(pos_h, pos_t, pos_r, pos_w, neg_h, neg_t, neg_r, neg_w,
                ent_emb, rel_emb)


# speedup vs baseline: 1.1255x; 1.1024x over previous
import functools

import jax
import jax.numpy as jnp
from jax import lax
from jax.experimental import pallas as pl
from jax.experimental.pallas import tpu as pltpu
from jax.experimental.pallas import tpu_sc as plsc

NC = 2
NS = 16
NW = NC * NS
L = 16

B = 16384
V = 1000000
R = 8
D = 64
W = 2 * D
MARGIN = 1.0
K = 8

S = B // NW
CHUNK = 64
NCHUNK = S // CHUNK
GROUPS = CHUNK // L
TP = L + 1


def _body(hidx, tidx, jhidx, jtidx, sp, pw, nw,
          ent2, relf,
          out_hbm,
          hidx_v, tidx_v, jhidx_v, jtidx_v,
          sp_v, pw_v, nw_v, relf_v,
          ph0, pt0, nh0, nt0, ph1, pt1, nh1, nt1,
          tsp, tsn, out_v, sem0, sem1):
    cid = lax.axis_index("c")
    sid = lax.axis_index("s")
    wid = sid * NC + cid
    base = wid * S

    stage = [
        pltpu.async_copy(hidx.at[pl.ds(base, S)], hidx_v, sem0),
        pltpu.async_copy(tidx.at[pl.ds(base, S)], tidx_v, sem0),
        pltpu.async_copy(jhidx.at[pl.ds(base, S)], jhidx_v, sem0),
        pltpu.async_copy(jtidx.at[pl.ds(base, S)], jtidx_v, sem0),
        pltpu.async_copy(sp.at[pl.ds(base * K, S * K)],
                         sp_v.at[pl.ds(0, S * K)], sem0),
        pltpu.async_copy(pw.at[pl.ds(base, S)], pw_v, sem0),
        pltpu.async_copy(nw.at[pl.ds(base, S)], nw_v, sem0),
        pltpu.async_copy(relf, relf_v, sem0),
    ]
    for dsc in stage:
        dsc.wait()

    bufs = ((ph0, pt0, nh0, nt0), (ph1, pt1, nh1, nt1))
    sems = (sem0, sem1)
    idxs = (hidx_v, tidx_v, jhidx_v, jtidx_v)

    def fire(c):
        p = c % 2
        return [
            pltpu.async_copy(ent2.at[idxs[k].at[pl.ds(c * CHUNK, CHUNK)]],
                             bufs[p][k], sems[p])
            for k in range(4)
        ]

    lanes = lax.iota(jnp.int32, L)
    cols = lanes * TP
    zf = jnp.zeros((L,), jnp.float32)

    pending = fire(0)
    total = zf
    for c in range(NCHUNK):
        nxt = fire(c + 1) if c + 1 < NCHUNK else []
        for dsc in pending:
            dsc.wait()
        phb, ptb, nhb, ntb = bufs[c % 2]

        def group_body(g, tot, phb=phb, ptb=ptb, nhb=nhb, ntb=ntb, c=c):
            goff = c * CHUNK + g * L

            def elem_body(i, _):
                e = g * L + i
                sv = sp_v[pl.ds((goff + i) * K, L)]
                oh = sv[0]
                ot = sv[1]
                ojh = sv[2]
                ojt = sv[3]
                pb = sv[4]
                nb = sv[5]
                ap = zf
                an = zf
                for f in range(D // L):
                    hp = phb[e, pl.ds(oh + f * L, L)]
                    tp = ptb[e, pl.ds(ot + f * L, L)]
                    rp = relf_v[pl.ds(pb + f * L, L)]
                    hn = nhb[e, pl.ds(ojh + f * L, L)]
                    tn = ntb[e, pl.ds(ojt + f * L, L)]
                    rn = relf_v[pl.ds(nb + f * L, L)]
                    ap = ap + jnp.abs(hp + rp - tp)
                    an = an + jnp.abs(hn + rn - tn)
                tsp[pl.ds(i * TP, L)] = ap
                tsn[pl.ds(i * TP, L)] = an
                return 0

            lax.fori_loop(0, L, elem_body, 0)

            ps = zf
            ns = zf
            for f in range(L):
                ps = ps + plsc.load_gather(tsp, [cols + f])
                ns = ns + plsc.load_gather(tsn, [cols + f])
            pwv = pw_v[pl.ds(goff, L)]
            nwv = nw_v[pl.ds(goff, L)]
            return tot + jnp.maximum(pwv * ps - nwv * ns + MARGIN, 0.0)

        total = lax.fori_loop(0, GROUPS, group_body, total)
        pending = nxt

    out_v[...] = total
    pltpu.sync_copy(out_v, out_hbm.at[pl.ds(wid * L, L)])


_rhine_sc = functools.partial(
    pl.kernel,
    out_type=jax.ShapeDtypeStruct((NW * L,), jnp.float32),
    mesh=plsc.VectorSubcoreMesh(core_axis_name="c", subcore_axis_name="s"),
    compiler_params=pltpu.CompilerParams(
        needs_layout_passes=False, use_tc_tiling_on_sc=True),
    scratch_types=[
        pltpu.VMEM((S,), jnp.int32),
        pltpu.VMEM((S,), jnp.int32),
        pltpu.VMEM((S,), jnp.int32),
        pltpu.VMEM((S,), jnp.int32),
        pltpu.VMEM((S * K + L,), jnp.int32),
        pltpu.VMEM((S,), jnp.float32),
        pltpu.VMEM((S,), jnp.float32),
        pltpu.VMEM((R * D,), jnp.float32),
        pltpu.VMEM((CHUNK, W), jnp.float32),
        pltpu.VMEM((CHUNK, W), jnp.float32),
        pltpu.VMEM((CHUNK, W), jnp.float32),
        pltpu.VMEM((CHUNK, W), jnp.float32),
        pltpu.VMEM((CHUNK, W), jnp.float32),
        pltpu.VMEM((CHUNK, W), jnp.float32),
        pltpu.VMEM((CHUNK, W), jnp.float32),
        pltpu.VMEM((CHUNK, W), jnp.float32),
        pltpu.VMEM((L * TP,), jnp.float32),
        pltpu.VMEM((L * TP,), jnp.float32),
        pltpu.VMEM((L,), jnp.float32),
        pltpu.SemaphoreType.DMA,
        pltpu.SemaphoreType.DMA,
    ],
)(_body)


@jax.jit
def _run(pos_h, pos_t, pos_r, pos_w, neg_h, neg_t, neg_r, neg_w,
         ent_emb, rel_emb):
    i32 = lambda x: x.astype(jnp.int32)
    ph, pt, nh, nt = i32(pos_h), i32(pos_t), i32(neg_h), i32(neg_t)
    sp = jnp.stack(
        [(ph & 1) * D, (pt & 1) * D, (nh & 1) * D, (nt & 1) * D,
         i32(pos_r) * D, i32(neg_r) * D,
         jnp.zeros((B,), jnp.int32), jnp.zeros((B,), jnp.int32)],
        axis=1).reshape(B * K)
    out = _rhine_sc(
        ph >> 1, pt >> 1, nh >> 1, nt >> 1,
        sp,
        pos_w.astype(jnp.float32), neg_w.astype(jnp.float32),
        ent_emb.astype(jnp.float32).reshape(V // 2, W),
        rel_emb.astype(jnp.float32).reshape(R * D),
    )
    return jnp.sum(out)


def kernel(pos_h, pos_t, pos_r, pos_w, neg_h, neg_t, neg_r, neg_w,
           ent_emb, rel_emb):
    return _run(pos_h, pos_t, pos_r, pos_w, neg_h, neg_t, neg_r, neg_w,
                ent_emb, rel_emb)


# speedup vs baseline: 1.1505x; 1.0222x over previous
import functools

import jax
import jax.numpy as jnp
from jax import lax
from jax.experimental import pallas as pl
from jax.experimental.pallas import tpu as pltpu
from jax.experimental.pallas import tpu_sc as plsc

NC = 2
NS = 16
NW = NC * NS
L = 16

B = 16384
V = 1000000
R = 8
D = 64
MARGIN = 1.0
K = 2

S = B // NW
CHUNK = 64
NCHUNK = S // CHUNK
GROUPS = CHUNK // L
TP = L + 1


def _body(hidx, tidx, jhidx, jtidx, sp, pw, nw,
          ent, relf,
          out_hbm,
          hidx_v, tidx_v, jhidx_v, jtidx_v,
          sp_v, pw_v, nw_v, relf_v,
          ph0, pt0, nh0, nt0, ph1, pt1, nh1, nt1,
          tsp, tsn, out_v, sem0, sem1):
    cid = lax.axis_index("c")
    sid = lax.axis_index("s")
    wid = sid * NC + cid
    base = wid * S

    stage = [
        pltpu.async_copy(hidx.at[pl.ds(base, S)], hidx_v, sem0),
        pltpu.async_copy(tidx.at[pl.ds(base, S)], tidx_v, sem0),
        pltpu.async_copy(jhidx.at[pl.ds(base, S)], jhidx_v, sem0),
        pltpu.async_copy(jtidx.at[pl.ds(base, S)], jtidx_v, sem0),
        pltpu.async_copy(sp.at[pl.ds(base * K, S * K)],
                         sp_v.at[pl.ds(0, S * K)], sem0),
        pltpu.async_copy(pw.at[pl.ds(base, S)], pw_v, sem0),
        pltpu.async_copy(nw.at[pl.ds(base, S)], nw_v, sem0),
        pltpu.async_copy(relf, relf_v, sem0),
    ]
    for dsc in stage:
        dsc.wait()

    bufs = ((ph0, pt0, nh0, nt0), (ph1, pt1, nh1, nt1))
    sems = (sem0, sem1)
    idxs = (hidx_v, tidx_v, jhidx_v, jtidx_v)

    def fire(c):
        p = c % 2
        return [
            pltpu.async_copy(ent.at[idxs[k].at[pl.ds(c * CHUNK, CHUNK)]],
                             bufs[p][k], sems[p])
            for k in range(4)
        ]

    lanes = lax.iota(jnp.int32, L)
    cols = lanes * TP
    zf = jnp.zeros((L,), jnp.float32)

    pending = fire(0)
    total = zf
    for c in range(NCHUNK):
        nxt = fire(c + 1) if c + 1 < NCHUNK else []
        for dsc in pending:
            dsc.wait()
        phb, ptb, nhb, ntb = bufs[c % 2]

        def group_body(g, tot, phb=phb, ptb=ptb, nhb=nhb, ntb=ntb, c=c):
            goff = c * CHUNK + g * L

            def elem_body(i, _):
                e = g * L + i
                sv = sp_v[pl.ds((goff + i) * K, L)]
                pb = sv[0]
                nb = sv[1]
                ap = zf
                an = zf
                for f in range(D // L):
                    hp = phb[e, pl.ds(f * L, L)]
                    tp = ptb[e, pl.ds(f * L, L)]
                    rp = relf_v[pl.ds(pb + f * L, L)]
                    hn = nhb[e, pl.ds(f * L, L)]
                    tn = ntb[e, pl.ds(f * L, L)]
                    rn = relf_v[pl.ds(nb + f * L, L)]
                    ap = ap + jnp.abs(hp + rp - tp)
                    an = an + jnp.abs(hn + rn - tn)
                tsp[pl.ds(i * TP, L)] = ap
                tsn[pl.ds(i * TP, L)] = an
                return 0

            lax.fori_loop(0, L, elem_body, 0)

            ps = zf
            ns = zf
            for f in range(L):
                ps = ps + plsc.load_gather(tsp, [cols + f])
                ns = ns + plsc.load_gather(tsn, [cols + f])
            pwv = pw_v[pl.ds(goff, L)]
            nwv = nw_v[pl.ds(goff, L)]
            return tot + jnp.maximum(pwv * ps - nwv * ns + MARGIN, 0.0)

        total = lax.fori_loop(0, GROUPS, group_body, total)
        pending = nxt

    out_v[...] = total
    pltpu.sync_copy(out_v, out_hbm.at[pl.ds(wid * L, L)])


_rhine_sc = functools.partial(
    pl.kernel,
    out_type=jax.ShapeDtypeStruct((NW * L,), jnp.float32),
    mesh=plsc.VectorSubcoreMesh(core_axis_name="c", subcore_axis_name="s"),
    compiler_params=pltpu.CompilerParams(
        needs_layout_passes=False, use_tc_tiling_on_sc=False),
    scratch_types=[
        pltpu.VMEM((S,), jnp.int32),
        pltpu.VMEM((S,), jnp.int32),
        pltpu.VMEM((S,), jnp.int32),
        pltpu.VMEM((S,), jnp.int32),
        pltpu.VMEM((S * K + L,), jnp.int32),
        pltpu.VMEM((S,), jnp.float32),
        pltpu.VMEM((S,), jnp.float32),
        pltpu.VMEM((R * D,), jnp.float32),
        pltpu.VMEM((CHUNK, D), jnp.float32),
        pltpu.VMEM((CHUNK, D), jnp.float32),
        pltpu.VMEM((CHUNK, D), jnp.float32),
        pltpu.VMEM((CHUNK, D), jnp.float32),
        pltpu.VMEM((CHUNK, D), jnp.float32),
        pltpu.VMEM((CHUNK, D), jnp.float32),
        pltpu.VMEM((CHUNK, D), jnp.float32),
        pltpu.VMEM((CHUNK, D), jnp.float32),
        pltpu.VMEM((L * TP,), jnp.float32),
        pltpu.VMEM((L * TP,), jnp.float32),
        pltpu.VMEM((L,), jnp.float32),
        pltpu.SemaphoreType.DMA,
        pltpu.SemaphoreType.DMA,
    ],
)(_body)


@jax.jit
def _run(pos_h, pos_t, pos_r, pos_w, neg_h, neg_t, neg_r, neg_w,
         ent_emb, rel_emb):
    i32 = lambda x: x.astype(jnp.int32)
    sp = jnp.stack([i32(pos_r) * D, i32(neg_r) * D], axis=1).reshape(B * K)
    out = _rhine_sc(
        i32(pos_h), i32(pos_t), i32(neg_h), i32(neg_t),
        sp,
        pos_w.astype(jnp.float32), neg_w.astype(jnp.float32),
        ent_emb,
        rel_emb.astype(jnp.float32).reshape(R * D),
    )
    return jnp.sum(out)


def kernel(pos_h, pos_t, pos_r, pos_w, neg_h, neg_t, neg_r, neg_w,
           ent_emb, rel_emb):
    return _run(pos_h, pos_t, pos_r, pos_w, neg_h, neg_t, neg_r, neg_w,
                ent_emb, rel_emb)
